# baseline (speedup 1.0000x reference)
import jax
import jax.numpy as jnp
from jax import lax
from jax.experimental import pallas as pl
from jax.experimental.pallas import tpu as pltpu, tpu_sc as plsc

B, K, A, V = 4096, 26, 100, 128
NC, NS = 2, 16
NW = NC * NS
SPW = B // NW
LANES = 16
VJ = V // LANES
G = 2
GK = G * K
GA = G * A
NG = SPW // G
NQ = NG // 2


def _body(values_hbm, idx_hbm, out_hbm, idx_v, vals_v, row_v,
          so0, so1, sin0, sin1):
    c = lax.axis_index("c")
    s = lax.axis_index("s")
    wid = s * NC + c
    base = wid * SPW

    pltpu.sync_copy(idx_hbm.at[pl.ds(base, SPW)], idx_v)

    zero16 = jnp.zeros((LANES,), jnp.float32)

    def zblk(a, acc):
        for j in range(VJ):
            row_v[0, a, pl.ds(j * LANES, LANES)] = zero16
            row_v[1, a, pl.ds(j * LANES, LANES)] = zero16
        return acc
    lax.fori_loop(0, GA, zblk, 0)

    sem_out = (so0, so1)
    sem_in = (sin0, sin1)

    def in_cp(slot, q):
        return pltpu.make_async_copy(
            values_hbm.at[pl.ds((base + q * 2 * G) * K, 2 * GK)],
            vals_v.at[slot], sem_in[slot])

    def out_cp(d, gi):
        return pltpu.make_async_copy(
            row_v.at[d], out_hbm.at[pl.ds((base + gi * G) * A, GA)],
            sem_out[d])

    def idx_lanes(si):
        iv0 = idx_v[si, pl.ds(0, LANES)]
        iv1 = idx_v[si, pl.ds(K - LANES, LANES)]
        return [iv0[k] if k < LANES else iv1[k - (K - LANES)]
                for k in range(K)]

    in_cp(0, 0).start()
    in_cp(1, 1).start()

    def step(g, carry):
        for qq in range(2):
            q = 2 * g + qq
            in_cp(qq, q).wait()
            for d in range(2):
                gi = 2 * q + d

                @pl.when(gi >= 2)
                def _():
                    out_cp(d, gi - 2).wait()
                    for gs in range(G):
                        for k, idx in enumerate(idx_lanes(2 * gi + gs - 4)):
                            a = idx + gs * A
                            for j in range(VJ):
                                row_v[d, a, pl.ds(j * LANES, LANES)] = zero16

                for gs in range(G):
                    src_row = (2 * d + gs) * K
                    for k, idx in enumerate(idx_lanes(2 * gi + gs)):
                        a = idx + gs * A
                        for j in range(VJ):
                            row_v[d, a, pl.ds(j * LANES, LANES)] = (
                                vals_v[qq, src_row + k,
                                       pl.ds(j * LANES, LANES)])

                out_cp(d, gi).start()

            @pl.when(q + 2 < NQ)
            def _():
                in_cp(qq, q + 2).start()
        return carry

    lax.fori_loop(0, NQ // 2, step, 0)

    out_cp(0, NG - 2).wait()
    out_cp(1, NG - 1).wait()


def kernel(values, attrib_idx):
    idx32 = attrib_idx.astype(jnp.int32)
    values2 = values.reshape(B * K, V)
    mesh = plsc.VectorSubcoreMesh(core_axis_name="c", subcore_axis_name="s")
    run = pl.kernel(
        _body,
        out_type=jax.ShapeDtypeStruct((B * A, V), jnp.float32),
        mesh=mesh,
        scratch_types=[
            pltpu.VMEM((SPW, K), jnp.int32),
            pltpu.VMEM((2, 2 * GK, V), jnp.float32),
            pltpu.VMEM((2, GA, V), jnp.float32),
        ] + [pltpu.SemaphoreType.DMA] * 4,
    )
    out = run(values2, idx32)
    return out.reshape(B, A, V)


# speedup vs baseline: 1.9806x; 1.9806x over previous
import jax
import jax.numpy as jnp
from jax import lax
from jax.experimental import pallas as pl
from jax.experimental.pallas import tpu as pltpu, tpu_sc as plsc

B, K, A, V = 4096, 26, 100, 128
NC, NS = 2, 16
NW = NC * NS
SPW = B // NW
LANES = 16
VJ = V // LANES


def _body(values_hbm, idx_hbm, out_hbm, idx_v, vals_v, row_v,
          sin0, sin1, so0, so1):
    c = lax.axis_index("c")
    s = lax.axis_index("s")
    wid = s * NC + c
    base = wid * SPW

    pltpu.sync_copy(idx_hbm.at[pl.ds(base, SPW)], idx_v)

    zero16 = jnp.zeros((LANES,), jnp.float32)

    def zblk(a, acc):
        for j in range(VJ):
            row_v[0, a, pl.ds(j * LANES, LANES)] = zero16
            row_v[1, a, pl.ds(j * LANES, LANES)] = zero16
        return acc
    lax.fori_loop(0, A, zblk, 0)

    sem_in = (sin0, sin1)
    sem_out = (so0, so1)

    def in_cp(d, b):
        return pltpu.make_async_copy(values_hbm.at[b], vals_v.at[d], sem_in[d])

    def out_cp(d, b):
        return pltpu.make_async_copy(row_v.at[d], out_hbm.at[b], sem_out[d])

    def idx_lanes(si):
        iv0 = idx_v[si, pl.ds(0, LANES)]
        iv1 = idx_v[si, pl.ds(K - LANES, LANES)]
        return [iv0[k] if k < LANES else iv1[k - (K - LANES)]
                for k in range(K)]

    in_cp(0, base).start()
    in_cp(1, base + 1).start()

    def step(g, carry):
        for d in range(2):
            si = 2 * g + d
            b = base + si

            @pl.when(si >= 2)
            def _():
                out_cp(d, b - 2).wait()
                pa = idx_lanes(si - 2)
                for j in range(VJ):
                    for k in range(K):
                        row_v[d, pa[k], pl.ds(j * LANES, LANES)] = zero16

            in_cp(d, b).wait()

            na = idx_lanes(si)
            for j in range(VJ):
                for k in range(K):
                    row_v[d, na[k], pl.ds(j * LANES, LANES)] = (
                        vals_v[d, k, pl.ds(j * LANES, LANES)])

            out_cp(d, b).start()

            @pl.when(si + 2 < SPW)
            def _():
                in_cp(d, b + 2).start()
        return carry

    lax.fori_loop(0, SPW // 2, step, 0)

    out_cp(0, base + SPW - 2).wait()
    out_cp(1, base + SPW - 1).wait()


def kernel(values, attrib_idx):
    idx32 = attrib_idx.astype(jnp.int32)
    mesh = plsc.VectorSubcoreMesh(core_axis_name="c", subcore_axis_name="s")
    run = pl.kernel(
        _body,
        out_type=jax.ShapeDtypeStruct((B, A, V), jnp.float32),
        mesh=mesh,
        scratch_types=[
            pltpu.VMEM((SPW, K), jnp.int32),
            pltpu.VMEM((2, K, V), jnp.float32),
            pltpu.VMEM((2, A, V), jnp.float32),
        ] + [pltpu.SemaphoreType.DMA] * 4,
    )
    return run(values, idx32)


# speedup vs baseline: 1.9831x; 1.0013x over previous
import jax
import jax.numpy as jnp
from jax import lax
from jax.experimental import pallas as pl
from jax.experimental.pallas import tpu as pltpu, tpu_sc as plsc

B, K, A, V = 4096, 26, 100, 128
NC, NS = 2, 16
NW = NC * NS
SPW = B // NW
LANES = 16
VJ = V // LANES


def _body(values_hbm, idx_hbm, out_hbm, idx_v, vals_v, row_v,
          sin0, sin1, so0, so1):
    c = lax.axis_index("c")
    s = lax.axis_index("s")
    wid = s * NC + c
    base = wid * SPW

    pltpu.sync_copy(idx_hbm.at[pl.ds(base, SPW)], idx_v)

    zero16 = jnp.zeros((LANES,), jnp.float32)

    def zblk(a, acc):
        for j in range(VJ):
            row_v[0, a, pl.ds(j * LANES, LANES)] = zero16
            row_v[1, a, pl.ds(j * LANES, LANES)] = zero16
        return acc
    lax.fori_loop(0, A, zblk, 0)

    sem_in = (sin0, sin1)
    sem_out = (so0, so1)

    def in_cp(d, b):
        return pltpu.make_async_copy(values_hbm.at[b], vals_v.at[d], sem_in[d])

    def out_cp(d, b):
        return pltpu.make_async_copy(row_v.at[d], out_hbm.at[b], sem_out[d])

    def idx_lanes(si):
        iv0 = idx_v[si, pl.ds(0, LANES)]
        iv1 = idx_v[si, pl.ds(K - LANES, LANES)]
        return [iv0[k] if k < LANES else iv1[k - (K - LANES)]
                for k in range(K)]

    in_cp(0, base).start()
    in_cp(1, base + 1).start()

    def step(g, carry):
        for d in range(2):
            si = 2 * g + d
            b = base + si

            @pl.when(si >= 2)
            def _():
                out_cp(d, b - 2).wait()
                pa = idx_lanes(si - 2)
                for j in range(VJ):
                    for k in range(K):
                        row_v[d, pa[k], pl.ds(j * LANES, LANES)] = zero16

            in_cp(d, b).wait()

            na = idx_lanes(si)
            for j in range(VJ):
                for k in range(K):
                    row_v[d, na[k], pl.ds(j * LANES, LANES)] = (
                        vals_v[d, k, pl.ds(j * LANES, LANES)])

            out_cp(d, b).start()

            @pl.when(si + 2 < SPW)
            def _():
                in_cp(d, b + 2).start()
        return carry

    lax.fori_loop(0, SPW // 2, step, 0)

    out_cp(0, base + SPW - 2).wait()
    out_cp(1, base + SPW - 1).wait()


def kernel(values, attrib_idx):
    idx32 = attrib_idx.astype(jnp.int32)
    mesh = plsc.VectorSubcoreMesh(core_axis_name="c", subcore_axis_name="s")
    run = pl.kernel(
        _body,
        out_type=jax.ShapeDtypeStruct((B, A, V), jnp.float32),
        mesh=mesh,
        compiler_params=pltpu.CompilerParams(use_tc_tiling_on_sc=True),
        scratch_types=[
            pltpu.VMEM((SPW, K), jnp.int32),
            pltpu.VMEM((2, K, V), jnp.float32),
            pltpu.VMEM((2, A, V), jnp.float32),
        ] + [pltpu.SemaphoreType.DMA] * 4,
    )
    return run(values, idx32)
